# baseline (speedup 1.0000x reference)
import jax
import jax.numpy as jnp
from jax import lax
from jax.experimental import pallas as pl
from jax.experimental.pallas import tpu as pltpu
from jax.experimental.pallas import tpu_sc as plsc

S = 30.0
M = 0.4

_B = 1024
_C = 100000

_NW = 32
_ROWS_PW = _B // _NW
_CHUNK = 10000
_T = (_ROWS_PW * _C) // _CHUNK
_NBUF = 4
_VREGS = _CHUNK // 16


def _sc_body(cos_hbm, fp_hbm, out_hbm, fp_v, vals_v, *bufs_and_sems):
    bufin = bufs_and_sems[:_NBUF]
    bufout = bufs_and_sems[_NBUF:2 * _NBUF]
    in_sems = bufs_and_sems[2 * _NBUF]
    out_sems = bufs_and_sems[2 * _NBUF + 1]
    fix_sem = bufs_and_sems[2 * _NBUF + 2]

    wid = lax.axis_index("s") * 2 + lax.axis_index("c")
    base_row = wid * _ROWS_PW
    base_elem = base_row * _C

    pltpu.sync_copy(fp_hbm.at[pl.ds(base_row, _ROWS_PW)], fp_v)

    def start_in(t, b):
        pltpu.async_copy(
            cos_hbm.at[pl.ds(base_elem + t * _CHUNK, _CHUNK)],
            bufin[b],
            in_sems.at[b],
        )

    def wait_in(t, b):
        pltpu.make_async_copy(
            cos_hbm.at[pl.ds(base_elem + t * _CHUNK, _CHUNK)],
            bufin[b],
            in_sems.at[b],
        ).wait()

    def start_out(t, b):
        pltpu.async_copy(
            bufout[b],
            out_hbm.at[pl.ds(base_elem + t * _CHUNK, _CHUNK)],
            out_sems.at[b],
        )

    def wait_out(t, b):
        pltpu.make_async_copy(
            bufout[b],
            out_hbm.at[pl.ds(base_elem + t * _CHUNK, _CHUNK)],
            out_sems.at[b],
        ).wait()

    for b in range(_NBUF):
        start_in(b, b)

    def round_body(g, _):
        for b in range(_NBUF):
            t = g * _NBUF + b
            wait_in(t, b)

            @pl.when(g > 0)
            def _():
                wait_out(t - _NBUF, b)

            def vec_body(j, _):
                sl = pl.ds(j * 16, 16)
                bufout[b][sl] = bufin[b][sl] * S
                return 0

            lax.fori_loop(0, _VREGS, vec_body, 0, unroll=25)

            start_out(t, b)

            @pl.when(t + _NBUF < _T)
            def _():
                start_in(t + _NBUF, b)
        return 0

    lax.fori_loop(0, _T // _NBUF, round_body, 0)

    for b in range(_NBUF):
        wait_out(_T - _NBUF + b, b)

    pltpu.async_copy(out_hbm.at[fp_v], vals_v, fix_sem).wait()
    for k in range(_ROWS_PW // 16):
        sl = pl.ds(k * 16, 16)
        vals_v[sl] = vals_v[sl] - jnp.float32(S * M)
    pltpu.async_copy(vals_v, out_hbm.at[fp_v], fix_sem).wait()


@jax.jit
def kernel(cosine, label):
    B, C = cosine.shape
    cos_flat = cosine.reshape(B * C)
    flatpos = (jnp.arange(B, dtype=jnp.int32) * C + label.astype(jnp.int32))

    mesh = plsc.VectorSubcoreMesh(core_axis_name="c", subcore_axis_name="s")
    out_flat = pl.kernel(
        _sc_body,
        mesh=mesh,
        out_type=jax.ShapeDtypeStruct((B * C,), jnp.float32),
        scratch_types=(
            [pltpu.VMEM((_ROWS_PW,), jnp.int32), pltpu.VMEM((_ROWS_PW,), jnp.float32)]
            + [pltpu.VMEM((_CHUNK,), jnp.float32) for _ in range(2 * _NBUF)]
            + [
                pltpu.SemaphoreType.DMA((_NBUF,)),
                pltpu.SemaphoreType.DMA((_NBUF,)),
                pltpu.SemaphoreType.DMA,
            ]
        ),
    )(cos_flat, flatpos)
    return out_flat.reshape(B, C)
